# Initial kernel scaffold; baseline (speedup 1.0000x reference)
import jax
import jax.numpy as jnp
from jax.experimental import pallas as pl


def kernel(input, target):
    raise NotImplementedError("write your pallas kernel here")


# speedup vs baseline: 8.0721x; 8.0721x over previous
import functools

import jax
import jax.numpy as jnp
from jax import lax
from jax.experimental import pallas as pl
from jax.experimental.pallas import tpu as pltpu
from jax.experimental.pallas import tpu_sc as plsc

N, S, NB = 64, 52, 3
M = N * S * S
NC, NS = 2, 16
NW = NC * NS
ROWS_W = M // NW
ITERS = ROWS_W // 16
IN_W = ROWS_W * 5 * NB
TG_W = ROWS_W * 5

_LOG1P = (2.554673020349618e-07, 0.9999670809438443, -0.49928504912226557,
          0.32722571497202635, -0.22316586411450423, 0.130833427976782,
          -0.05243753706207599, 0.01000928961639147)


def _log1p_poly(t):
    acc = jnp.full_like(t, _LOG1P[7])
    for c in _LOG1P[6::-1]:
        acc = acc * t + c
    return acc


def _softplus(x):
    return jnp.maximum(x, 0.0) + _log1p_poly(jnp.exp(-jnp.abs(x)))


def _sigmoid(x):
    return 1.0 / (1.0 + jnp.exp(-x))


def _loss_body(in_hbm, tg_hbm, out_hbm, in_v, tg_v, acc_v):
    wid = lax.axis_index("s") * NC + lax.axis_index("c")
    pltpu.sync_copy(in_hbm.at[pl.ds(wid * IN_W, IN_W)], in_v)
    pltpu.sync_copy(tg_hbm.at[pl.ds(wid * TG_W, TG_W)], tg_v)

    lane = lax.iota(jnp.int32, 16)
    lane15 = lane * 15
    lane5 = lane * 5

    def body(i, accs):
        s_no1, s_no2, s_obj, s_bb, n_obj = accs
        ib = i * 240 + lane15
        tb = i * 80 + lane5

        tconf = plsc.load_gather(tg_v, [tb])
        tx = plsc.load_gather(tg_v, [tb + 1])
        ty = plsc.load_gather(tg_v, [tb + 2])
        tw = plsc.load_gather(tg_v, [tb + 3])
        th = plsc.load_gather(tg_v, [tb + 4])
        objf = jnp.where(tconf > 0.0, 1.0, 0.0)

        tx1, tx2 = tx - tw * 0.5, tx + tw * 0.5
        ty1, ty2 = ty - th * 0.5, ty + th * 0.5
        t_area = (tx2 - tx1) * (ty2 - ty1)

        ious, logits, boxes, sps = [], [], [], []
        for b in range(NB):
            l = plsc.load_gather(in_v, [ib + 5 * b])
            pxc = _sigmoid(plsc.load_gather(in_v, [ib + 5 * b + 1]))
            pyc = _sigmoid(plsc.load_gather(in_v, [ib + 5 * b + 2]))
            pw = _sigmoid(plsc.load_gather(in_v, [ib + 5 * b + 3]))
            ph = _sigmoid(plsc.load_gather(in_v, [ib + 5 * b + 4]))
            px1, px2 = pxc - pw * 0.5, pxc + pw * 0.5
            py1, py2 = pyc - ph * 0.5, pyc + ph * 0.5
            ix = jnp.maximum(jnp.minimum(px2, tx2) - jnp.maximum(px1, tx1), 0.0)
            iy = jnp.maximum(jnp.minimum(py2, ty2) - jnp.maximum(py1, ty1), 0.0)
            inter = ix * iy
            p_area = (px2 - px1) * (py2 - py1)
            ious.append(inter / (p_area + t_area - inter + 1e-6))
            logits.append(l)
            boxes.append((pxc, pyc, pw, ph))
            sps.append(_softplus(l))

        m0 = (ious[0] >= ious[1]) & (ious[0] >= ious[2])
        m1 = jnp.logical_not(m0) & (ious[1] >= ious[2])

        def sel(a0, a1, a2):
            return jnp.where(m0, a0, jnp.where(m1, a1, a2))

        spsum = sps[0] + sps[1] + sps[2]
        sp_r = sel(*sps)
        l_r = sel(*logits)
        dx = sel(boxes[0][0], boxes[1][0], boxes[2][0]) - tx
        dy = sel(boxes[0][1], boxes[1][1], boxes[2][1]) - ty
        dw = sel(boxes[0][2], boxes[1][2], boxes[2][2]) - tw
        dh = sel(boxes[0][3], boxes[1][3], boxes[2][3]) - th
        mse = dx * dx + dy * dy + dw * dw + dh * dh

        noobjf = 1.0 - objf
        return (s_no1 + spsum * noobjf,
                s_no2 + (spsum - sp_r) * objf,
                s_obj + (sp_r - l_r) * objf,
                s_bb + mse * objf,
                n_obj + objf)

    zero = jnp.zeros((16,), jnp.float32)
    accs = lax.fori_loop(0, ITERS, body, (zero, zero, zero, zero, zero))
    for k in range(5):
        acc_v[pl.ds(k * 16, 16)] = accs[k]
    pltpu.sync_copy(acc_v, out_hbm.at[pl.ds(wid * 80, 80)])


@jax.jit
def kernel(input, target):
    in1d = input.reshape(-1)
    tg1d = target.reshape(-1)
    mesh = plsc.VectorSubcoreMesh(core_axis_name="c", subcore_axis_name="s",
                                  num_cores=NC, num_subcores=NS)
    partials = pl.kernel(
        _loss_body,
        out_type=jax.ShapeDtypeStruct((NW * 80,), jnp.float32),
        mesh=mesh,
        compiler_params=pltpu.CompilerParams(needs_layout_passes=False),
        scratch_types=[
            pltpu.VMEM((IN_W,), jnp.float32),
            pltpu.VMEM((TG_W,), jnp.float32),
            pltpu.VMEM((80,), jnp.float32),
        ],
    )(in1d, tg1d)

    p = partials.reshape(NW, 5, 16).sum(axis=(0, 2))
    s_no1, s_no2, s_obj, s_bb, n_obj = p[0], p[1], p[2], p[3], p[4]
    n_noobj = M - n_obj
    loss_noobj = s_no1 / (n_noobj * NB) + s_no2 / (n_obj * (NB - 1))
    loss_bbox = s_bb / (n_obj * 4.0)
    loss_obj = s_obj / n_obj
    return (loss_obj + loss_bbox + loss_noobj, loss_noobj, loss_bbox, loss_obj)


# speedup vs baseline: 8.1120x; 1.0049x over previous
import jax
import jax.numpy as jnp
from jax import lax
from jax.experimental import pallas as pl
from jax.experimental.pallas import tpu as pltpu
from jax.experimental.pallas import tpu_sc as plsc

N, S, NB = 64, 52, 3
M = N * S * S
NC, NS = 2, 16
NW = NC * NS
ROWS_W = M // NW
NCHUNK = 13
ROWS_C = ROWS_W // NCHUNK
ITERS_C = ROWS_C // 16
IN_C = ROWS_C * 5 * NB
TG_C = ROWS_C * 5

_LOG1P = (2.554673020349618e-07, 0.9999670809438443, -0.49928504912226557,
          0.32722571497202635, -0.22316586411450423, 0.130833427976782,
          -0.05243753706207599, 0.01000928961639147)


def _log1p_poly(t):
    acc = jnp.full_like(t, _LOG1P[7])
    for c in _LOG1P[6::-1]:
        acc = acc * t + c
    return acc


def _softplus(x):
    return jnp.maximum(x, 0.0) + _log1p_poly(jnp.exp(-jnp.abs(x)))


def _sigmoid(x):
    return 1.0 / (1.0 + jnp.exp(-x))


def _loss_body(in_hbm, tg_hbm, out_hbm, in_v0, in_v1, tg_v0, tg_v1, acc_v,
               in_sem0, in_sem1, tg_sem0, tg_sem1):
    wid = lax.axis_index("s") * NC + lax.axis_index("c")
    base_in = wid * (ROWS_W * 5 * NB)
    base_tg = wid * (ROWS_W * 5)
    in_bufs, tg_bufs = (in_v0, in_v1), (tg_v0, tg_v1)
    in_sems, tg_sems = (in_sem0, in_sem1), (tg_sem0, tg_sem1)

    def copy_in(c, b):
        return pltpu.make_async_copy(
            in_hbm.at[pl.ds(base_in + c * IN_C, IN_C)], in_bufs[b], in_sems[b])

    def copy_tg(c, b):
        return pltpu.make_async_copy(
            tg_hbm.at[pl.ds(base_tg + c * TG_C, TG_C)], tg_bufs[b], tg_sems[b])

    lane = lax.iota(jnp.int32, 16)
    lane15 = lane * 15
    lane5 = lane * 5

    def body(args):
        i, b, accs = args
        s_no1, s_no2, s_obj, s_bb, n_obj = accs
        ib = i * 240 + lane15
        tb = i * 80 + lane5
        inb, tgb = in_bufs[b], tg_bufs[b]

        tconf = plsc.load_gather(tgb, [tb])
        tx = plsc.load_gather(tgb, [tb + 1])
        ty = plsc.load_gather(tgb, [tb + 2])
        tw = plsc.load_gather(tgb, [tb + 3])
        th = plsc.load_gather(tgb, [tb + 4])
        objf = jnp.where(tconf > 0.0, 1.0, 0.0)

        tx1, tx2 = tx - tw * 0.5, tx + tw * 0.5
        ty1, ty2 = ty - th * 0.5, ty + th * 0.5
        t_area = (tx2 - tx1) * (ty2 - ty1)

        inters, unions, logits, boxes, sps = [], [], [], [], []
        for bb in range(NB):
            l = plsc.load_gather(inb, [ib + 5 * bb])
            pxc = _sigmoid(plsc.load_gather(inb, [ib + 5 * bb + 1]))
            pyc = _sigmoid(plsc.load_gather(inb, [ib + 5 * bb + 2]))
            pw = _sigmoid(plsc.load_gather(inb, [ib + 5 * bb + 3]))
            ph = _sigmoid(plsc.load_gather(inb, [ib + 5 * bb + 4]))
            px1, px2 = pxc - pw * 0.5, pxc + pw * 0.5
            py1, py2 = pyc - ph * 0.5, pyc + ph * 0.5
            ix = jnp.maximum(jnp.minimum(px2, tx2) - jnp.maximum(px1, tx1), 0.0)
            iy = jnp.maximum(jnp.minimum(py2, ty2) - jnp.maximum(py1, ty1), 0.0)
            inter = ix * iy
            p_area = (px2 - px1) * (py2 - py1)
            inters.append(inter)
            unions.append(p_area + t_area - inter + 1e-6)
            logits.append(l)
            boxes.append((pxc, pyc, pw, ph))
            sps.append(_softplus(l))

        m0 = ((inters[0] * unions[1] >= inters[1] * unions[0]) &
              (inters[0] * unions[2] >= inters[2] * unions[0]))
        m1 = (jnp.logical_not(m0) &
              (inters[1] * unions[2] >= inters[2] * unions[1]))

        def sel(a0, a1, a2):
            return jnp.where(m0, a0, jnp.where(m1, a1, a2))

        spsum = sps[0] + sps[1] + sps[2]
        sp_r = sel(*sps)
        l_r = sel(*logits)
        dx = sel(boxes[0][0], boxes[1][0], boxes[2][0]) - tx
        dy = sel(boxes[0][1], boxes[1][1], boxes[2][1]) - ty
        dw = sel(boxes[0][2], boxes[1][2], boxes[2][2]) - tw
        dh = sel(boxes[0][3], boxes[1][3], boxes[2][3]) - th
        mse = dx * dx + dy * dy + dw * dw + dh * dh

        noobjf = 1.0 - objf
        return (s_no1 + spsum * noobjf,
                s_no2 + (spsum - sp_r) * objf,
                s_obj + (sp_r - l_r) * objf,
                s_bb + mse * objf,
                n_obj + objf)

    zero = jnp.zeros((16,), jnp.float32)
    accs = (zero, zero, zero, zero, zero)

    copy_in(0, 0).start()
    copy_tg(0, 0).start()
    for c in range(NCHUNK):
        b = c % 2
        if c + 1 < NCHUNK:
            copy_in(c + 1, 1 - b).start()
            copy_tg(c + 1, 1 - b).start()
        copy_in(c, b).wait()
        copy_tg(c, b).wait()

        def chunk_body(i, accs):
            return body((i, b, accs))

        accs = lax.fori_loop(0, ITERS_C, chunk_body, accs)

    for k in range(5):
        acc_v[pl.ds(k * 16, 16)] = accs[k]
    pltpu.sync_copy(acc_v, out_hbm.at[pl.ds(wid * 80, 80)])


@jax.jit
def kernel(input, target):
    in1d = input.reshape(-1)
    tg1d = target.reshape(-1)
    mesh = plsc.VectorSubcoreMesh(core_axis_name="c", subcore_axis_name="s",
                                  num_cores=NC, num_subcores=NS)
    partials = pl.kernel(
        _loss_body,
        out_type=jax.ShapeDtypeStruct((NW * 80,), jnp.float32),
        mesh=mesh,
        compiler_params=pltpu.CompilerParams(needs_layout_passes=False),
        scratch_types=[
            pltpu.VMEM((IN_C,), jnp.float32),
            pltpu.VMEM((IN_C,), jnp.float32),
            pltpu.VMEM((TG_C,), jnp.float32),
            pltpu.VMEM((TG_C,), jnp.float32),
            pltpu.VMEM((80,), jnp.float32),
            pltpu.SemaphoreType.DMA,
            pltpu.SemaphoreType.DMA,
            pltpu.SemaphoreType.DMA,
            pltpu.SemaphoreType.DMA,
        ],
    )(in1d, tg1d)

    p = partials.reshape(NW, 5, 16).sum(axis=(0, 2))
    s_no1, s_no2, s_obj, s_bb, n_obj = p[0], p[1], p[2], p[3], p[4]
    n_noobj = M - n_obj
    loss_noobj = s_no1 / (n_noobj * NB) + s_no2 / (n_obj * (NB - 1))
    loss_bbox = s_bb / (n_obj * 4.0)
    loss_obj = s_obj / n_obj
    return (loss_obj + loss_bbox + loss_noobj, loss_noobj, loss_bbox, loss_obj)
